# baseline (speedup 1.0000x reference)
import jax
import jax.numpy as jnp
from jax import lax
from jax.experimental import pallas as pl
from jax.experimental.pallas import tpu as pltpu
from jax.experimental.pallas import tpu_sc as plsc

B, S, D_MODEL = 1, 2048, 768
N_EXPERTS, TOP_K = 8, 2
D_FFN = int(D_MODEL * 2.0)
LN_EPS = 1e-5

TILE = 512
P = S * TOP_K + N_EXPERTS * TILE
NT = P // TILE
CHUNK = 256
N_CHUNKS = S // CHUNK


def _router_kernel(x_ref, rw_ref, dest_ref, wtb_ref, te_ref):
    xb = x_ref[...]

    logits = lax.dot_general(
        xb, rw_ref[...], (((1,), (1,)), ((), ())),
        preferred_element_type=jnp.float32)
    lane = lax.broadcasted_iota(jnp.int32, (S, N_EXPERTS), 1)
    max1 = jnp.max(logits, axis=1, keepdims=True)
    arg1 = jnp.min(jnp.where(logits == max1, lane, N_EXPERTS), axis=1,
                   keepdims=True)
    masked = jnp.where(lane == arg1, -jnp.inf, logits)
    max2 = jnp.max(masked, axis=1, keepdims=True)
    arg2 = jnp.min(jnp.where(masked == max2, lane, N_EXPERTS), axis=1,
                   keepdims=True)
    w1 = jax.nn.sigmoid(max1 - max2)

    wtb_ref[...] = jnp.concatenate(
        [jnp.broadcast_to(w1, (S, 16)),
         jnp.broadcast_to(1.0 - w1, (S, 16))], axis=1)

    m0 = (lane == arg1).astype(jnp.float32)
    m1 = (lane == arg2).astype(jnp.float32)

    r = lax.broadcasted_iota(jnp.int32, (CHUNK, CHUNK), 0)
    c = lax.broadcasted_iota(jnp.int32, (CHUNK, CHUNK), 1)
    tri = (r >= c).astype(jnp.float32)
    c0_parts, c1_parts = [], []
    carry0 = jnp.zeros((1, N_EXPERTS), jnp.float32)
    for ch in range(N_CHUNKS):
        blk = m0[ch * CHUNK:(ch + 1) * CHUNK, :]
        c0_parts.append(lax.dot_general(
            tri, blk, (((1,), (0,)), ((), ())),
            preferred_element_type=jnp.float32) + carry0)
        carry0 = carry0 + jnp.sum(blk, axis=0, keepdims=True)
    carry1 = carry0
    for ch in range(N_CHUNKS):
        blk = m1[ch * CHUNK:(ch + 1) * CHUNK, :]
        c1_parts.append(lax.dot_general(
            tri, blk, (((1,), (0,)), ((), ())),
            preferred_element_type=jnp.float32) + carry1)
        carry1 = carry1 + jnp.sum(blk, axis=0, keepdims=True)
    c0 = jnp.concatenate(c0_parts, axis=0)
    c1 = jnp.concatenate(c1_parts, axis=0)
    counts = carry1

    padded = jnp.ceil(counts / TILE) * TILE
    scols = [jnp.zeros((1, 1), jnp.float32)]
    acc = jnp.zeros((1, 1), jnp.float32)
    for e in range(N_EXPERTS - 1):
        acc = acc + padded[:, e:e + 1]
        scols.append(acc)
    start = jnp.concatenate(scols, axis=1)

    startb = jnp.broadcast_to(start, (S, N_EXPERTS))
    rank0 = jnp.sum(c0 * m0, axis=1, keepdims=True) - 1.0
    rank1 = jnp.sum(c1 * m1, axis=1, keepdims=True) - 1.0
    dest0 = jnp.sum(startb * m0, axis=1, keepdims=True) + rank0
    dest1 = jnp.sum(startb * m1, axis=1, keepdims=True) + rank1
    dz = jnp.zeros((1, S), jnp.float32)
    dest_ref[...] = jnp.concatenate(
        [dest0.reshape(1, S), dest1.reshape(1, S),
         dz, dz, dz, dz, dz, dz], axis=0).astype(jnp.int32)

    j = lax.broadcasted_iota(jnp.int32, (TILE, N_EXPERTS), 0).astype(
        jnp.float32) * TILE
    sb = jnp.broadcast_to(start, (TILE, N_EXPERTS))
    te = jnp.sum((j >= sb).astype(jnp.int32), axis=1, keepdims=True) - 1
    te = jnp.clip(te, 0, N_EXPERTS - 1)
    te_ref[...] = jnp.broadcast_to(te, (TILE, N_EXPERTS))


F_BLK = D_FFN // 2


def _ffn_kernel(te_ref, xs_ref, wg_ref, wu_ref, wd_ref, ys_ref):
    del te_ref
    f = pl.program_id(1)
    xb = xs_ref[...]
    gate = lax.dot_general(
        xb, wg_ref[0], (((1,), (1,)), ((), ())),
        preferred_element_type=jnp.float32)
    up = lax.dot_general(
        xb, wu_ref[0], (((1,), (1,)), ((), ())),
        preferred_element_type=jnp.float32)
    h = (jax.nn.silu(gate) * up).astype(jnp.bfloat16)
    eo = lax.dot_general(
        h, wd_ref[0], (((1,), (1,)), ((), ())),
        preferred_element_type=jnp.float32)

    @pl.when(f == 0)
    def _():
        ys_ref[...] = eo

    @pl.when(f == 1)
    def _():
        ys_ref[...] += eo


def _ln_kernel(cb_ref, g_ref, b_ref, out_ref):
    o = cb_ref[...]
    mean = jnp.mean(o, axis=1, keepdims=True)
    cen = o - mean
    var = jnp.mean(cen * cen, axis=1, keepdims=True)
    out_ref[...] = cen * lax.rsqrt(var + LN_EPS) * g_ref[...] + b_ref[...]


def _make_scatter_kernel(n_workers):
    rows_per_w = (S * TOP_K) // n_workers
    slabs = S // rows_per_w

    def scatter_kernel(x_hbm, dest_hbm, xs_hbm, idx_v, rows_v, sem):
        wid = lax.axis_index("s") * 2 + lax.axis_index("c")
        k = wid // slabs
        tbase = (wid % slabs) * rows_per_w
        pltpu.sync_copy(dest_hbm.at[k, pl.ds(tbase, rows_per_w)], idx_v)
        pltpu.sync_copy(x_hbm.at[pl.ds(tbase, rows_per_w)], rows_v)
        pltpu.async_copy(rows_v, xs_hbm.at[idx_v], sem).wait()

    return scatter_kernel, rows_per_w


def _make_combine_kernel(n_workers):
    tok_per_w = S // n_workers
    n_ch = D_MODEL // 16

    def combine_kernel(ys_hbm, dest_hbm, wtb_hbm, cb_hbm,
                       idx0_v, idx1_v, y0_v, y1_v, wtb_v, sem):
        wid = lax.axis_index("s") * 2 + lax.axis_index("c")
        tbase = wid * tok_per_w
        pltpu.sync_copy(dest_hbm.at[0, pl.ds(tbase, tok_per_w)], idx0_v)
        pltpu.sync_copy(dest_hbm.at[1, pl.ds(tbase, tok_per_w)], idx1_v)
        pltpu.sync_copy(wtb_hbm.at[pl.ds(tbase, tok_per_w)], wtb_v)
        cp0 = pltpu.async_copy(ys_hbm.at[idx0_v], y0_v, sem)
        cp1 = pltpu.async_copy(ys_hbm.at[idx1_v], y1_v, sem)
        cp0.wait()
        cp1.wait()

        def row_body(r, _):
            w0 = wtb_v[r, 0:16]
            w1 = wtb_v[r, 16:32]
            for ch in range(n_ch):
                sl = pl.ds(ch * 16, 16)
                y0_v[r, sl] = y0_v[r, sl] * w0 + y1_v[r, sl] * w1
            return _

        lax.fori_loop(0, tok_per_w, row_body, 0)
        pltpu.sync_copy(y0_v, cb_hbm.at[pl.ds(tbase, tok_per_w)])

    return combine_kernel, tok_per_w


def kernel(x, router_w, w_gate, w_up, w_down, ln_gamma, ln_beta):
    x2 = x.reshape(S, D_MODEL)
    gamma = ln_gamma.reshape(1, D_MODEL)
    beta = ln_beta.reshape(1, D_MODEL)

    dest8, wtb, te_out = pl.pallas_call(
        _router_kernel,
        grid=(1,),
        in_specs=[
            pl.BlockSpec((S, D_MODEL), lambda i: (0, 0)),
            pl.BlockSpec((N_EXPERTS, D_MODEL), lambda i: (0, 0)),
        ],
        out_specs=[
            pl.BlockSpec((8, S), lambda i: (0, 0)),
            pl.BlockSpec((S, 32), lambda i: (0, 0)),
            pl.BlockSpec((TILE, N_EXPERTS), lambda i: (0, 0)),
        ],
        out_shape=[
            jax.ShapeDtypeStruct((8, S), jnp.int32),
            jax.ShapeDtypeStruct((S, 32), jnp.float32),
            jax.ShapeDtypeStruct((TILE, N_EXPERTS), jnp.int32),
        ],
    )(x2, router_w)
    te = te_out[:NT, 0]

    info = plsc.get_sparse_core_info()
    n_workers = info.num_cores * info.num_subcores

    scatter_body, rows_per_w = _make_scatter_kernel(n_workers)
    mesh = plsc.VectorSubcoreMesh(core_axis_name="c", subcore_axis_name="s")
    xs = pl.kernel(
        scatter_body,
        mesh=mesh,
        out_type=jax.ShapeDtypeStruct((P, D_MODEL), jnp.float32),
        scratch_types=[
            pltpu.VMEM((rows_per_w,), jnp.int32),
            pltpu.VMEM((rows_per_w, D_MODEL), jnp.float32),
            pltpu.SemaphoreType.DMA,
        ],
    )(x2, dest8)

    ys = pl.pallas_call(
        _ffn_kernel,
        grid_spec=pltpu.PrefetchScalarGridSpec(
            num_scalar_prefetch=1,
            grid=(NT, 2),
            in_specs=[
                pl.BlockSpec((TILE, D_MODEL), lambda t, f, te: (t, 0)),
                pl.BlockSpec((1, F_BLK, D_MODEL),
                             lambda t, f, te: (te[t], f, 0)),
                pl.BlockSpec((1, F_BLK, D_MODEL),
                             lambda t, f, te: (te[t], f, 0)),
                pl.BlockSpec((1, D_MODEL, F_BLK),
                             lambda t, f, te: (te[t], 0, f)),
            ],
            out_specs=pl.BlockSpec((TILE, D_MODEL), lambda t, f, te: (t, 0)),
        ),
        out_shape=jax.ShapeDtypeStruct((P, D_MODEL), jnp.float32),
    )(te, xs, w_gate, w_up, w_down)

    combine_body, tok_per_w = _make_combine_kernel(n_workers)
    comb = pl.kernel(
        combine_body,
        mesh=plsc.VectorSubcoreMesh(core_axis_name="c", subcore_axis_name="s"),
        out_type=jax.ShapeDtypeStruct((S, D_MODEL), jnp.float32),
        scratch_types=[
            pltpu.VMEM((tok_per_w,), jnp.int32),
            pltpu.VMEM((tok_per_w,), jnp.int32),
            pltpu.VMEM((tok_per_w, D_MODEL), jnp.float32),
            pltpu.VMEM((tok_per_w, D_MODEL), jnp.float32),
            pltpu.VMEM((tok_per_w, 32), jnp.float32),
            pltpu.SemaphoreType.DMA,
        ],
    )(ys, dest8, wtb)

    out = pl.pallas_call(
        _ln_kernel,
        grid=(1,),
        in_specs=[
            pl.BlockSpec((S, D_MODEL), lambda i: (0, 0)),
            pl.BlockSpec((1, D_MODEL), lambda i: (0, 0)),
            pl.BlockSpec((1, D_MODEL), lambda i: (0, 0)),
        ],
        out_specs=pl.BlockSpec((S, D_MODEL), lambda i: (0, 0)),
        out_shape=jax.ShapeDtypeStruct((S, D_MODEL), jnp.float32),
    )(comb, gamma, beta)
    return out.reshape(B, S, D_MODEL)


# speedup vs baseline: 1.1058x; 1.1058x over previous
import jax
import jax.numpy as jnp
from jax import lax
from jax.experimental import pallas as pl
from jax.experimental.pallas import tpu as pltpu
from jax.experimental.pallas import tpu_sc as plsc

B, S, D_MODEL = 1, 2048, 768
N_EXPERTS, TOP_K = 8, 2
D_FFN = int(D_MODEL * 2.0)
LN_EPS = 1e-5

TILE = 512
P = S * TOP_K + N_EXPERTS * TILE
NT = P // TILE
CHUNK = 256
N_CHUNKS = S // CHUNK


def _router_kernel(x_ref, rw_ref, dest_ref, wtb_ref, te_ref):
    xb = x_ref[...]

    logits = lax.dot_general(
        xb, rw_ref[...], (((1,), (1,)), ((), ())),
        preferred_element_type=jnp.float32)
    lane = lax.broadcasted_iota(jnp.int32, (S, N_EXPERTS), 1)
    max1 = jnp.max(logits, axis=1, keepdims=True)
    arg1 = jnp.min(jnp.where(logits == max1, lane, N_EXPERTS), axis=1,
                   keepdims=True)
    masked = jnp.where(lane == arg1, -jnp.inf, logits)
    max2 = jnp.max(masked, axis=1, keepdims=True)
    arg2 = jnp.min(jnp.where(masked == max2, lane, N_EXPERTS), axis=1,
                   keepdims=True)
    w1 = jax.nn.sigmoid(max1 - max2)

    wtb_ref[...] = jnp.concatenate(
        [jnp.broadcast_to(w1, (S, 16)),
         jnp.broadcast_to(1.0 - w1, (S, 16))], axis=1)

    m0 = (lane == arg1).astype(jnp.float32)
    m1 = (lane == arg2).astype(jnp.float32)

    r = lax.broadcasted_iota(jnp.int32, (CHUNK, CHUNK), 0)
    c = lax.broadcasted_iota(jnp.int32, (CHUNK, CHUNK), 1)
    tri = (r >= c).astype(jnp.float32)
    c0_parts, c1_parts = [], []
    carry0 = jnp.zeros((1, N_EXPERTS), jnp.float32)
    for ch in range(N_CHUNKS):
        blk = m0[ch * CHUNK:(ch + 1) * CHUNK, :]
        c0_parts.append(lax.dot_general(
            tri, blk, (((1,), (0,)), ((), ())),
            preferred_element_type=jnp.float32) + carry0)
        carry0 = carry0 + jnp.sum(blk, axis=0, keepdims=True)
    carry1 = carry0
    for ch in range(N_CHUNKS):
        blk = m1[ch * CHUNK:(ch + 1) * CHUNK, :]
        c1_parts.append(lax.dot_general(
            tri, blk, (((1,), (0,)), ((), ())),
            preferred_element_type=jnp.float32) + carry1)
        carry1 = carry1 + jnp.sum(blk, axis=0, keepdims=True)
    c0 = jnp.concatenate(c0_parts, axis=0)
    c1 = jnp.concatenate(c1_parts, axis=0)
    counts = carry1

    padded = jnp.ceil(counts / TILE) * TILE
    scols = [jnp.zeros((1, 1), jnp.float32)]
    acc = jnp.zeros((1, 1), jnp.float32)
    for e in range(N_EXPERTS - 1):
        acc = acc + padded[:, e:e + 1]
        scols.append(acc)
    start = jnp.concatenate(scols, axis=1)

    startb = jnp.broadcast_to(start, (S, N_EXPERTS))
    rank0 = jnp.sum(c0 * m0, axis=1, keepdims=True) - 1.0
    rank1 = jnp.sum(c1 * m1, axis=1, keepdims=True) - 1.0
    dest0 = jnp.sum(startb * m0, axis=1, keepdims=True) + rank0
    dest1 = jnp.sum(startb * m1, axis=1, keepdims=True) + rank1
    dz = jnp.zeros((1, S), jnp.float32)
    dest_ref[...] = jnp.concatenate(
        [dest0.reshape(1, S), dest1.reshape(1, S),
         dz, dz, dz, dz, dz, dz], axis=0).astype(jnp.int32)

    j = lax.broadcasted_iota(jnp.int32, (TILE, N_EXPERTS), 0).astype(
        jnp.float32) * TILE
    sb = jnp.broadcast_to(start, (TILE, N_EXPERTS))
    te = jnp.sum((j >= sb).astype(jnp.int32), axis=1, keepdims=True) - 1
    te = jnp.clip(te, 0, N_EXPERTS - 1)
    te_ref[...] = jnp.broadcast_to(te, (TILE, N_EXPERTS))


def _ffn_kernel(te_ref, xs_ref, wg_ref, wu_ref, wd_ref, ys_ref):
    del te_ref
    xb = xs_ref[...]
    gate = lax.dot_general(
        xb, wg_ref[0], (((1,), (1,)), ((), ())),
        preferred_element_type=jnp.float32)
    up = lax.dot_general(
        xb, wu_ref[0], (((1,), (1,)), ((), ())),
        preferred_element_type=jnp.float32)
    h = (jax.nn.silu(gate) * up).astype(jnp.bfloat16)
    ys_ref[...] = lax.dot_general(
        h, wd_ref[0], (((1,), (1,)), ((), ())),
        preferred_element_type=jnp.float32)


def _ln_kernel(cb_ref, g_ref, b_ref, out_ref):
    o = cb_ref[...]
    mean = jnp.mean(o, axis=1, keepdims=True)
    cen = o - mean
    var = jnp.mean(cen * cen, axis=1, keepdims=True)
    out_ref[...] = cen * lax.rsqrt(var + LN_EPS) * g_ref[...] + b_ref[...]


def _make_scatter_kernel(n_workers):
    rows_per_w = (S * TOP_K) // n_workers
    slabs = S // rows_per_w

    def scatter_kernel(x_hbm, dest_hbm, xs_hbm, idx_v, rows_v, sem):
        wid = lax.axis_index("s") * 2 + lax.axis_index("c")
        k = wid // slabs
        tbase = (wid % slabs) * rows_per_w
        pltpu.sync_copy(dest_hbm.at[k, pl.ds(tbase, rows_per_w)], idx_v)
        pltpu.sync_copy(x_hbm.at[pl.ds(tbase, rows_per_w)], rows_v)
        pltpu.async_copy(rows_v, xs_hbm.at[idx_v], sem).wait()

    return scatter_kernel, rows_per_w


def _make_combine_kernel(n_workers):
    tok_per_w = S // n_workers
    n_ch = D_MODEL // 16

    def combine_kernel(ys_hbm, dest_hbm, wtb_hbm, cb_hbm,
                       idx0_v, idx1_v, y0_v, y1_v, wtb_v, sem):
        wid = lax.axis_index("s") * 2 + lax.axis_index("c")
        tbase = wid * tok_per_w
        pltpu.sync_copy(dest_hbm.at[0, pl.ds(tbase, tok_per_w)], idx0_v)
        pltpu.sync_copy(dest_hbm.at[1, pl.ds(tbase, tok_per_w)], idx1_v)
        pltpu.sync_copy(wtb_hbm.at[pl.ds(tbase, tok_per_w)], wtb_v)
        cp0 = pltpu.async_copy(ys_hbm.at[idx0_v], y0_v, sem)
        cp1 = pltpu.async_copy(ys_hbm.at[idx1_v], y1_v, sem)
        cp0.wait()
        cp1.wait()

        def row_body(r, _):
            w0 = wtb_v[r, 0:16]
            w1 = wtb_v[r, 16:32]
            for ch in range(n_ch):
                sl = pl.ds(ch * 16, 16)
                y0_v[r, sl] = y0_v[r, sl] * w0 + y1_v[r, sl] * w1
            return _

        lax.fori_loop(0, tok_per_w, row_body, 0)
        pltpu.sync_copy(y0_v, cb_hbm.at[pl.ds(tbase, tok_per_w)])

    return combine_kernel, tok_per_w


def kernel(x, router_w, w_gate, w_up, w_down, ln_gamma, ln_beta):
    x2 = x.reshape(S, D_MODEL)
    gamma = ln_gamma.reshape(1, D_MODEL)
    beta = ln_beta.reshape(1, D_MODEL)

    dest8, wtb, te_out = pl.pallas_call(
        _router_kernel,
        grid=(1,),
        in_specs=[
            pl.BlockSpec((S, D_MODEL), lambda i: (0, 0)),
            pl.BlockSpec((N_EXPERTS, D_MODEL), lambda i: (0, 0)),
        ],
        out_specs=[
            pl.BlockSpec((8, S), lambda i: (0, 0)),
            pl.BlockSpec((S, 32), lambda i: (0, 0)),
            pl.BlockSpec((TILE, N_EXPERTS), lambda i: (0, 0)),
        ],
        out_shape=[
            jax.ShapeDtypeStruct((8, S), jnp.int32),
            jax.ShapeDtypeStruct((S, 32), jnp.float32),
            jax.ShapeDtypeStruct((TILE, N_EXPERTS), jnp.int32),
        ],
    )(x2, router_w)
    te = te_out[:NT, 0]

    info = plsc.get_sparse_core_info()
    n_workers = info.num_cores * info.num_subcores

    scatter_body, rows_per_w = _make_scatter_kernel(n_workers)
    mesh = plsc.VectorSubcoreMesh(core_axis_name="c", subcore_axis_name="s")
    xs = pl.kernel(
        scatter_body,
        mesh=mesh,
        out_type=jax.ShapeDtypeStruct((P, D_MODEL), jnp.float32),
        scratch_types=[
            pltpu.VMEM((rows_per_w,), jnp.int32),
            pltpu.VMEM((rows_per_w, D_MODEL), jnp.float32),
            pltpu.SemaphoreType.DMA,
        ],
    )(x2, dest8)

    ys = pl.pallas_call(
        _ffn_kernel,
        grid_spec=pltpu.PrefetchScalarGridSpec(
            num_scalar_prefetch=1,
            grid=(NT,),
            in_specs=[
                pl.BlockSpec((TILE, D_MODEL), lambda t, te: (t, 0)),
                pl.BlockSpec((1, D_FFN, D_MODEL), lambda t, te: (te[t], 0, 0)),
                pl.BlockSpec((1, D_FFN, D_MODEL), lambda t, te: (te[t], 0, 0)),
                pl.BlockSpec((1, D_MODEL, D_FFN), lambda t, te: (te[t], 0, 0)),
            ],
            out_specs=pl.BlockSpec((TILE, D_MODEL), lambda t, te: (t, 0)),
        ),
        out_shape=jax.ShapeDtypeStruct((P, D_MODEL), jnp.float32),
    )(te, xs, w_gate, w_up, w_down)

    combine_body, tok_per_w = _make_combine_kernel(n_workers)
    comb = pl.kernel(
        combine_body,
        mesh=plsc.VectorSubcoreMesh(core_axis_name="c", subcore_axis_name="s"),
        out_type=jax.ShapeDtypeStruct((S, D_MODEL), jnp.float32),
        scratch_types=[
            pltpu.VMEM((tok_per_w,), jnp.int32),
            pltpu.VMEM((tok_per_w,), jnp.int32),
            pltpu.VMEM((tok_per_w, D_MODEL), jnp.float32),
            pltpu.VMEM((tok_per_w, D_MODEL), jnp.float32),
            pltpu.VMEM((tok_per_w, 32), jnp.float32),
            pltpu.SemaphoreType.DMA,
        ],
    )(ys, dest8, wtb)

    out = pl.pallas_call(
        _ln_kernel,
        grid=(1,),
        in_specs=[
            pl.BlockSpec((S, D_MODEL), lambda i: (0, 0)),
            pl.BlockSpec((1, D_MODEL), lambda i: (0, 0)),
            pl.BlockSpec((1, D_MODEL), lambda i: (0, 0)),
        ],
        out_specs=pl.BlockSpec((S, D_MODEL), lambda i: (0, 0)),
        out_shape=jax.ShapeDtypeStruct((S, D_MODEL), jnp.float32),
    )(comb, gamma, beta)
    return out.reshape(B, S, D_MODEL)
